# Initial kernel scaffold; baseline (speedup 1.0000x reference)
import jax
import jax.numpy as jnp
from jax.experimental import pallas as pl


def kernel(input_ids, table, W_enc, b_enc, W_dec, b_dec, W_out, b_out):
    raise NotImplementedError("write your pallas kernel here")


# speedup vs baseline: 43.3106x; 43.3106x over previous
import functools

import jax
import jax.numpy as jnp
from jax.experimental import pallas as pl
from jax.experimental.pallas import tpu as pltpu

BETA = 0.9
THETA = 1.0
ALPHA = 4.0

TCHUNK = 256


def _gif_scan(i_ref, o_ref, v_ref, nb):
    sub = 8 // nb

    def tile_step(k, v):
        tile = i_ref[pl.ds(8 * k, 8), :]
        outs = []
        for j in range(sub):
            i_t = tile[j * nb:(j + 1) * nb, :]
            v_new = BETA * v + i_t
            s = jax.nn.sigmoid(ALPHA * (v_new - THETA))
            outs.append(s)
            v = v_new * (1.0 - s)
        o_ref[pl.ds(8 * k, 8), :] = jnp.concatenate(outs, axis=0)
        return v

    v_ref[...] = jax.lax.fori_loop(0, TCHUNK * nb // 8, tile_step,
                                   v_ref[...])


def _zone_kernel(ids_ref, table_ref, we_ref, be_ref, wd_ref, bd_ref,
                 wo_ref, bo_ref, out_ref, ibuf, sbuf, v1_ref, v2_ref,
                 *, nb, vpad):
    @pl.when(pl.program_id(0) == 0)
    def _init():
        v1_ref[...] = jnp.zeros_like(v1_ref)
        v2_ref[...] = jnp.zeros_like(v2_ref)

    rows = ids_ref.shape[0]
    onehot = (ids_ref[...] == jax.lax.broadcasted_iota(
        jnp.int32, (rows, vpad), 1)).astype(jnp.float32)
    emb = jnp.dot(onehot, table_ref[...], preferred_element_type=jnp.float32)

    ibuf[...] = jnp.dot(emb, we_ref[...],
                        preferred_element_type=jnp.float32) + be_ref[...]
    _gif_scan(ibuf, sbuf, v1_ref, nb)

    ibuf[...] = jnp.dot(sbuf[...], wd_ref[...],
                        preferred_element_type=jnp.float32) + bd_ref[...]
    _gif_scan(ibuf, sbuf, v2_ref, nb)

    out_ref[...] = jnp.dot(sbuf[...], wo_ref[...],
                           preferred_element_type=jnp.float32) + bo_ref[...]


def kernel(input_ids, table, W_enc, b_enc, W_dec, b_dec, W_out, b_out):
    nb, t = input_ids.shape
    vocab, embed = table.shape
    hidden = W_enc.shape[1]
    vpad = (vocab + 127) // 128 * 128
    rows = TCHUNK * nb
    grid = t // TCHUNK

    ids2d = input_ids.astype(jnp.int32).T.reshape(t * nb, 1)
    table_p = jnp.pad(table, ((0, vpad - vocab), (0, 0)))
    wo_p = jnp.pad(W_out, ((0, 0), (0, vpad - vocab)))
    bo_p = jnp.pad(b_out, (0, vpad - vocab))[None, :]

    out = pl.pallas_call(
        functools.partial(_zone_kernel, nb=nb, vpad=vpad),
        grid=(grid,),
        in_specs=[
            pl.BlockSpec((rows, 1), lambda i: (i, 0)),
            pl.BlockSpec((vpad, embed), lambda i: (0, 0)),
            pl.BlockSpec((embed, hidden), lambda i: (0, 0)),
            pl.BlockSpec((1, hidden), lambda i: (0, 0)),
            pl.BlockSpec((hidden, embed), lambda i: (0, 0)),
            pl.BlockSpec((1, embed), lambda i: (0, 0)),
            pl.BlockSpec((embed, vpad), lambda i: (0, 0)),
            pl.BlockSpec((1, vpad), lambda i: (0, 0)),
        ],
        out_specs=pl.BlockSpec((rows, vpad), lambda i: (i, 0)),
        out_shape=jax.ShapeDtypeStruct((t * nb, vpad), jnp.float32),
        scratch_shapes=[
            pltpu.VMEM((rows, hidden), jnp.float32),
            pltpu.VMEM((rows, hidden), jnp.float32),
            pltpu.VMEM((nb, hidden), jnp.float32),
            pltpu.VMEM((nb, embed), jnp.float32),
        ],
    )(ids2d, table_p, W_enc, b_enc[None, :], W_dec, b_dec[None, :],
      wo_p, bo_p)

    return out.reshape(t, nb, vpad).transpose(1, 0, 2)[:, :, :vocab]


# speedup vs baseline: 43.3532x; 1.0010x over previous
import functools

import jax
import jax.numpy as jnp
from jax.experimental import pallas as pl
from jax.experimental.pallas import tpu as pltpu

BETA = 0.9
THETA = 1.0
ALPHA = 4.0

TCHUNK = 256


def _gif_scan(i_ref, o_ref, v_ref, nb):
    sub = 8 // nb

    def tile_step(k, v):
        tile = i_ref[pl.ds(8 * k, 8), :]
        outs = []
        for j in range(sub):
            i_t = tile[j * nb:(j + 1) * nb, :]
            v_new = BETA * v + i_t
            u = jnp.exp(ALPHA * THETA - ALPHA * v_new)
            s = 1.0 / (1.0 + u)
            outs.append(s)
            v = v_new - v_new * s
        o_ref[pl.ds(8 * k, 8), :] = jnp.concatenate(outs, axis=0)
        return v

    v_ref[...] = jax.lax.fori_loop(0, TCHUNK * nb // 8, tile_step,
                                   v_ref[...])


def _zone_kernel(ids_ref, table_ref, we_ref, be_ref, wd_ref, bd_ref,
                 wo_ref, bo_ref, out_ref, ibuf, sbuf, v1_ref, v2_ref,
                 *, nb, vpad):
    @pl.when(pl.program_id(0) == 0)
    def _init():
        v1_ref[...] = jnp.zeros_like(v1_ref)
        v2_ref[...] = jnp.zeros_like(v2_ref)

    rows = ids_ref.shape[0]
    onehot = (ids_ref[...] == jax.lax.broadcasted_iota(
        jnp.int32, (rows, vpad), 1)).astype(jnp.float32)
    emb = jnp.dot(onehot, table_ref[...], preferred_element_type=jnp.float32)

    ibuf[...] = jnp.dot(emb, we_ref[...],
                        preferred_element_type=jnp.float32) + be_ref[...]
    _gif_scan(ibuf, sbuf, v1_ref, nb)

    ibuf[...] = jnp.dot(sbuf[...], wd_ref[...],
                        preferred_element_type=jnp.float32) + bd_ref[...]
    _gif_scan(ibuf, sbuf, v2_ref, nb)

    out_ref[...] = jnp.dot(sbuf[...], wo_ref[...],
                           preferred_element_type=jnp.float32) + bo_ref[...]


def kernel(input_ids, table, W_enc, b_enc, W_dec, b_dec, W_out, b_out):
    nb, t = input_ids.shape
    vocab, embed = table.shape
    hidden = W_enc.shape[1]
    vpad = (vocab + 127) // 128 * 128
    rows = TCHUNK * nb
    grid = t // TCHUNK

    ids2d = input_ids.astype(jnp.int32).T.reshape(t * nb, 1)
    table_p = jnp.pad(table, ((0, vpad - vocab), (0, 0)))
    wo_p = jnp.pad(W_out, ((0, 0), (0, vpad - vocab)))
    bo_p = jnp.pad(b_out, (0, vpad - vocab))[None, :]

    out = pl.pallas_call(
        functools.partial(_zone_kernel, nb=nb, vpad=vpad),
        grid=(grid,),
        in_specs=[
            pl.BlockSpec((rows, 1), lambda i: (i, 0)),
            pl.BlockSpec((vpad, embed), lambda i: (0, 0)),
            pl.BlockSpec((embed, hidden), lambda i: (0, 0)),
            pl.BlockSpec((1, hidden), lambda i: (0, 0)),
            pl.BlockSpec((hidden, embed), lambda i: (0, 0)),
            pl.BlockSpec((1, embed), lambda i: (0, 0)),
            pl.BlockSpec((embed, vpad), lambda i: (0, 0)),
            pl.BlockSpec((1, vpad), lambda i: (0, 0)),
        ],
        out_specs=pl.BlockSpec((rows, vpad), lambda i: (i, 0)),
        out_shape=jax.ShapeDtypeStruct((t * nb, vpad), jnp.float32),
        scratch_shapes=[
            pltpu.VMEM((rows, hidden), jnp.float32),
            pltpu.VMEM((rows, hidden), jnp.float32),
            pltpu.VMEM((nb, hidden), jnp.float32),
            pltpu.VMEM((nb, embed), jnp.float32),
        ],
    )(ids2d, table_p, W_enc, b_enc[None, :], W_dec, b_dec[None, :],
      wo_p, bo_p)

    return out.reshape(t, nb, vpad).transpose(1, 0, 2)[:, :, :vocab]


# speedup vs baseline: 52.1299x; 1.2024x over previous
import functools

import jax
import jax.numpy as jnp
from jax.experimental import pallas as pl
from jax.experimental.pallas import tpu as pltpu

BETA = 0.9
THETA = 1.0
ALPHA = 4.0

TCHUNK = 256


def _gif_scan(i_ref, o_ref, v_ref, nb):
    sub = 8 // nb

    ha = 0.5 * ALPHA

    def tile_step(k, v):
        tile = i_ref[pl.ds(8 * k, 8), :]
        wt = ha * tile - (ha * THETA)
        outs = []
        for j in range(sub):
            i_t = tile[j * nb:(j + 1) * nb, :]
            w_t = wt[j * nb:(j + 1) * nb, :]
            x = (ha * BETA) * v + w_t
            tt = jnp.tanh(x)
            v_new = BETA * v + i_t
            h = 0.5 * v_new
            outs.append(0.5 * tt + 0.5)
            v = h - h * tt
        o_ref[pl.ds(8 * k, 8), :] = jnp.concatenate(outs, axis=0)
        return v

    v_ref[...] = jax.lax.fori_loop(0, TCHUNK * nb // 8, tile_step,
                                   v_ref[...])


def _zone_kernel(ids_ref, table_ref, we_ref, be_ref, wd_ref, bd_ref,
                 wo_ref, bo_ref, out_ref, ibuf, sbuf, v1_ref, v2_ref,
                 *, nb, vpad):
    @pl.when(pl.program_id(0) == 0)
    def _init():
        v1_ref[...] = jnp.zeros_like(v1_ref)
        v2_ref[...] = jnp.zeros_like(v2_ref)

    rows = ids_ref.shape[0]
    onehot = (ids_ref[...] == jax.lax.broadcasted_iota(
        jnp.int32, (rows, vpad), 1)).astype(jnp.float32)
    emb = jnp.dot(onehot, table_ref[...], preferred_element_type=jnp.float32)

    ibuf[...] = jnp.dot(emb, we_ref[...],
                        preferred_element_type=jnp.float32) + be_ref[...]
    _gif_scan(ibuf, sbuf, v1_ref, nb)

    ibuf[...] = jnp.dot(sbuf[...], wd_ref[...],
                        preferred_element_type=jnp.float32) + bd_ref[...]
    _gif_scan(ibuf, sbuf, v2_ref, nb)

    out_ref[...] = jnp.dot(sbuf[...], wo_ref[...],
                           preferred_element_type=jnp.float32) + bo_ref[...]


def kernel(input_ids, table, W_enc, b_enc, W_dec, b_dec, W_out, b_out):
    nb, t = input_ids.shape
    vocab, embed = table.shape
    hidden = W_enc.shape[1]
    vpad = (vocab + 127) // 128 * 128
    rows = TCHUNK * nb
    grid = t // TCHUNK

    ids2d = input_ids.astype(jnp.int32).T.reshape(t * nb, 1)
    table_p = jnp.pad(table, ((0, vpad - vocab), (0, 0)))
    wo_p = jnp.pad(W_out, ((0, 0), (0, vpad - vocab)))
    bo_p = jnp.pad(b_out, (0, vpad - vocab))[None, :]

    out = pl.pallas_call(
        functools.partial(_zone_kernel, nb=nb, vpad=vpad),
        grid=(grid,),
        in_specs=[
            pl.BlockSpec((rows, 1), lambda i: (i, 0)),
            pl.BlockSpec((vpad, embed), lambda i: (0, 0)),
            pl.BlockSpec((embed, hidden), lambda i: (0, 0)),
            pl.BlockSpec((1, hidden), lambda i: (0, 0)),
            pl.BlockSpec((hidden, embed), lambda i: (0, 0)),
            pl.BlockSpec((1, embed), lambda i: (0, 0)),
            pl.BlockSpec((embed, vpad), lambda i: (0, 0)),
            pl.BlockSpec((1, vpad), lambda i: (0, 0)),
        ],
        out_specs=pl.BlockSpec((rows, vpad), lambda i: (i, 0)),
        out_shape=jax.ShapeDtypeStruct((t * nb, vpad), jnp.float32),
        scratch_shapes=[
            pltpu.VMEM((rows, hidden), jnp.float32),
            pltpu.VMEM((rows, hidden), jnp.float32),
            pltpu.VMEM((nb, hidden), jnp.float32),
            pltpu.VMEM((nb, embed), jnp.float32),
        ],
    )(ids2d, table_p, W_enc, b_enc[None, :], W_dec, b_dec[None, :],
      wo_p, bo_p)

    return out.reshape(t, nb, vpad).transpose(1, 0, 2)[:, :, :vocab]


# speedup vs baseline: 74.5498x; 1.4301x over previous
import functools

import jax
import jax.numpy as jnp
from jax.experimental import pallas as pl
from jax.experimental.pallas import tpu as pltpu

BETA = 0.9
THETA = 1.0
ALPHA = 4.0

TCHUNK = 256


def _gif_scan(i_ref, o_ref, v_ref, nb, deinterleave=False):
    sub = 8 // nb

    ha = 0.5 * ALPHA

    if not deinterleave:
        def tile_step(k, v):
            tile = i_ref[pl.ds(8 * k, 8), :]
            wt = ha * tile - (ha * THETA)
            outs = []
            for j in range(sub):
                i_t = tile[j * nb:(j + 1) * nb, :]
                w_t = wt[j * nb:(j + 1) * nb, :]
                x = (ha * BETA) * v + w_t
                tt = jnp.tanh(x)
                v_new = BETA * v + i_t
                h = 0.5 * v_new
                outs.append(0.5 * tt + 0.5)
                v = h - h * tt
            o_ref[pl.ds(8 * k, 8), :] = jnp.concatenate(outs, axis=0)
            return v

        v_ref[...] = jax.lax.fori_loop(0, TCHUNK * nb // 8, tile_step,
                                       v_ref[...])
        return

    def tile_step16(k, v):
        tiles = [i_ref[pl.ds(16 * k, 8), :], i_ref[pl.ds(16 * k + 8, 8), :]]
        wts = [ha * tl - (ha * THETA) for tl in tiles]
        outs = []
        for j in range(2 * sub):
            half, jj = divmod(j, sub)
            i_t = tiles[half][jj * nb:(jj + 1) * nb, :]
            w_t = wts[half][jj * nb:(jj + 1) * nb, :]
            x = (ha * BETA) * v + w_t
            tt = jnp.tanh(x)
            v_new = BETA * v + i_t
            h = 0.5 * v_new
            outs.append(0.5 * tt + 0.5)
            v = h - h * tt
        for b in range(nb):
            blk = jnp.concatenate([o[b:b + 1, :] for o in outs], axis=0)
            o_ref[pl.ds(b * TCHUNK + 8 * k, 8), :] = blk
        return v

    v_ref[...] = jax.lax.fori_loop(0, TCHUNK * nb // 16, tile_step16,
                                   v_ref[...])


def _zone_kernel(ids_ref, table_ref, we_ref, be_ref, wd_ref, bd_ref,
                 wo_ref, bo_ref, out_ref, ibuf, sbuf, v1_ref, v2_ref,
                 *, nb, vpad):
    @pl.when(pl.program_id(0) == 0)
    def _init():
        v1_ref[...] = jnp.zeros_like(v1_ref)
        v2_ref[...] = jnp.zeros_like(v2_ref)

    rows = ids_ref.shape[0]
    onehot = (ids_ref[...] == jax.lax.broadcasted_iota(
        jnp.int32, (rows, vpad), 1)).astype(jnp.float32)
    emb = jnp.dot(onehot, table_ref[...], preferred_element_type=jnp.float32)

    ibuf[...] = jnp.dot(emb, we_ref[...],
                        preferred_element_type=jnp.float32) + be_ref[...]
    _gif_scan(ibuf, sbuf, v1_ref, nb)

    ibuf[...] = jnp.dot(sbuf[...], wd_ref[...],
                        preferred_element_type=jnp.float32) + bd_ref[...]
    _gif_scan(ibuf, sbuf, v2_ref, nb, deinterleave=True)

    logits = jnp.dot(sbuf[...], wo_ref[...],
                     preferred_element_type=jnp.float32) + bo_ref[...]
    out_ref[...] = logits.reshape(out_ref.shape)


def kernel(input_ids, table, W_enc, b_enc, W_dec, b_dec, W_out, b_out):
    nb, t = input_ids.shape
    vocab, embed = table.shape
    hidden = W_enc.shape[1]
    vpad = (vocab + 127) // 128 * 128
    rows = TCHUNK * nb
    grid = t // TCHUNK

    ids2d = input_ids.astype(jnp.int32).T.reshape(t * nb, 1)
    table_p = jnp.pad(table, ((0, vpad - vocab), (0, 0)))

    out = pl.pallas_call(
        functools.partial(_zone_kernel, nb=nb, vpad=vpad),
        grid=(grid,),
        in_specs=[
            pl.BlockSpec((rows, 1), lambda i: (i, 0)),
            pl.BlockSpec((vpad, embed), lambda i: (0, 0)),
            pl.BlockSpec((embed, hidden), lambda i: (0, 0)),
            pl.BlockSpec((1, hidden), lambda i: (0, 0)),
            pl.BlockSpec((hidden, embed), lambda i: (0, 0)),
            pl.BlockSpec((1, embed), lambda i: (0, 0)),
            pl.BlockSpec((embed, vocab), lambda i: (0, 0)),
            pl.BlockSpec((1, vocab), lambda i: (0, 0)),
        ],
        out_specs=pl.BlockSpec((nb, TCHUNK, vocab), lambda i: (0, i, 0)),
        out_shape=jax.ShapeDtypeStruct((nb, t, vocab), jnp.float32),
        scratch_shapes=[
            pltpu.VMEM((rows, hidden), jnp.float32),
            pltpu.VMEM((rows, hidden), jnp.float32),
            pltpu.VMEM((nb, hidden), jnp.float32),
            pltpu.VMEM((nb, embed), jnp.float32),
        ],
    )(ids2d, table_p, W_enc, b_enc[None, :], W_dec, b_dec[None, :],
      W_out, b_out[None, :])

    return out
